# baseline (speedup 1.0000x reference)
import functools

import jax
import jax.numpy as jnp
from jax import lax
from jax.experimental import pallas as pl
from jax.experimental.pallas import tpu as pltpu
from jax.experimental.pallas import tpu_sc as plsc

_N = 100000
_NP = 100352
_F = 128
_E = 64
_EP = 128
_B = 4096
_D = 32
_R = 3
_K = 16
_SCALE = 0.5 / _K

_NW = 32
_CB = _B // _NW
_ROWS = _R * _CB
_NBLK = _ROWS // 16
_GRP = 4
_NG = _ROWS // _GRP
_RING = 8
_SG = 128
_NSG = _ROWS * _D // _SG


def _batcher_pairs(n):
    pairs = []

    def merge(lo, m, r):
        step = r * 2
        if step < m:
            merge(lo, m, step)
            merge(lo + r, m, step)
            for i in range(lo + r, lo + m - r, step):
                pairs.append((i, i + r))
        else:
            pairs.append((lo, lo + r))

    def sort(lo, m):
        if m > 1:
            h = m // 2
            sort(lo, h)
            sort(lo + h, h)
            merge(lo, m, 1)

    sort(0, n)
    return pairs


_CE_PAIRS = _batcher_pairs(_K)


def _tc_body(f_ref, w_ref, wl_ref, fw_ref, fs_ref):
    f = f_ref[...]
    fw_ref[...] = jnp.dot(f, w_ref[...], preferred_element_type=jnp.float32)
    i = pl.program_id(0)
    fs = jnp.dot(f, wl_ref[...], preferred_element_type=jnp.float32)[:, 0]
    fs_ref[pl.ds(i * _TC_BLK, _TC_BLK)] = fs


_TC_BLK = 2048
_tc_precompute = pl.pallas_call(
    _tc_body,
    grid=(_NP // _TC_BLK,),
    in_specs=[
        pl.BlockSpec((_TC_BLK, _F), lambda i: (i, 0)),
        pl.BlockSpec((_F, _EP), lambda i: (0, 0)),
        pl.BlockSpec((_F, 1), lambda i: (0, 0)),
    ],
    out_specs=[
        pl.BlockSpec((_TC_BLK, _EP), lambda i: (i, 0)),
        pl.BlockSpec((_NP,), lambda i: (0,)),
    ],
    out_shape=[
        jax.ShapeDtypeStruct((_NP, _EP), jnp.float32),
        jax.ShapeDtypeStruct((_NP,), jnp.float32),
    ],
)


_mesh = plsc.VectorSubcoreMesh(
    core_axis_name="c", subcore_axis_name="s", num_cores=2, num_subcores=16
)


@functools.partial(
    pl.kernel,
    out_type=jax.ShapeDtypeStruct((_B, _E), jnp.float32),
    mesh=_mesh,
    compiler_params=pltpu.CompilerParams(
        needs_layout_passes=False, use_tc_tiling_on_sc=False
    ),
    scratch_types=[
        pltpu.VMEM((_CB,), jnp.int32),
        pltpu.VMEM((_CB,), jnp.int32),
        pltpu.VMEM((_CB,), jnp.float32),
        pltpu.VMEM((_ROWS * _D,), jnp.int32),
        pltpu.VMEM((_ROWS * _D,), jnp.float32),
        pltpu.VMEM((_ROWS * _K,), jnp.int32),
        pltpu.VMEM((_CB, _E), jnp.float32),
        pltpu.VMEM((_RING, _GRP * _K, _E), jnp.float32),
        pltpu.SemaphoreType.DMA,
        pltpu.SemaphoreType.DMA,
        pltpu.SemaphoreType.DMA,
        pltpu.SemaphoreType.DMA,
        pltpu.SemaphoreType.DMA,
        pltpu.SemaphoreType.DMA,
        pltpu.SemaphoreType.DMA,
        pltpu.SemaphoreType.DMA,
        pltpu.SemaphoreType.DMA,
    ],
)
def _sc_agg(nodes_hbm, neighf_hbm, fw_hbm, fs_hbm, out_hbm,
            nodesv, nodes2v, csv, idxv, nsv, selv, outv, gbuf,
            sem0, semA, semB, semC, semD, semE, semF, semG, semH):
    wid = lax.axis_index("s") * 2 + lax.axis_index("c")
    base = wid * _CB

    pltpu.sync_copy(nodes_hbm.at[pl.ds(base, _CB)], nodesv)
    for r in range(_R):
        pltpu.sync_copy(
            neighf_hbm.at[pl.ds(r * _B * _D + base * _D, _CB * _D)],
            idxv.at[pl.ds(r * _CB * _D, _CB * _D)],
        )

    cs_cp = pltpu.async_copy(fs_hbm.at[nodesv], csv, sem0)
    def _dbl_body(t, carry):
        v = nodesv[pl.ds(t * 16, 16)]
        nodes2v[pl.ds(t * 16, 16)] = v + v
        return carry
    lax.fori_loop(0, _CB // 16, _dbl_body, 0)
    ch_cp0 = pltpu.async_copy(
        fw_hbm.at[nodes2v.at[pl.ds(0, _CB // 2)]], gbuf.at[0], sem0)
    ch_cp1 = pltpu.async_copy(
        fw_hbm.at[nodes2v.at[pl.ds(_CB // 2, _CB // 2)]], gbuf.at[1], sem0)

    with jax.named_scope("sc_scores"):
        qs = _ROWS * _D // 4
        ns_cps = [
            pltpu.async_copy(
                fs_hbm.at[idxv.at[pl.ds(q * qs, qs)]],
                nsv.at[pl.ds(q * qs, qs)], sem)
            for q, sem in enumerate((semA, semB, semC, semD))
        ]
        cs_cp.wait()
        for cp in ns_cps:
            cp.wait()

    def _sel_body(blk, carry):
        rr = blk * 16 + lax.iota(jnp.int32, 16)
        b = lax.rem(rr, jnp.int32(_CB))
        cs = plsc.load_gather(csv, [b])
        base32 = rr * _D
        base16 = rr * _K
        keys = []
        vals = []
        for j in range(_D):
            s = plsc.load_gather(nsv, [base32 + j])
            keys.append(jnp.abs(s - cs))
            vals.append(plsc.load_gather(idxv, [base32 + j]))
        for half in (0, _K):
            for (i, j) in _CE_PAIRS:
                a, bb = half + i, half + j
                c = keys[a] <= keys[bb]
                ka, kb = keys[a], keys[bb]
                va, vb = vals[a], vals[bb]
                keys[a] = jnp.where(c, ka, kb)
                keys[bb] = jnp.where(c, kb, ka)
                vals[a] = jnp.where(c, va, vb)
                vals[bb] = jnp.where(c, vb, va)
        for j in range(_K):
            a, bb = j, _K + (_K - 1 - j)
            c = keys[a] <= keys[bb]
            sel = jnp.where(c, vals[a], vals[bb])
            plsc.store_scatter(selv, [base16 + j], sel + sel)
        return carry

    with jax.named_scope("sc_select"):
        lax.fori_loop(0, _NBLK, _sel_body, 0)
        ch_cp0.wait()
        ch_cp1.wait()
        def _ch_body(b, carry):
            for c in range(_E // 16):
                outv[b, pl.ds(16 * c, 16)] = gbuf[0, b, pl.ds(16 * c, 16)]
                outv[b + _CB // 2, pl.ds(16 * c, 16)] = (
                    gbuf[1, b, pl.ds(16 * c, 16)])
            return carry
        lax.fori_loop(0, _CB // 2, _ch_body, 0)

    sems = (semA, semB, semC, semD, semE, semF, semG, semH)

    def _issue_rows(g, slot, sem):
        pltpu.async_copy(
            fw_hbm.at[selv.at[pl.ds(g * _GRP * _K, _GRP * _K)]],
            gbuf.at[slot], sem)

    def _wait_rows(g, slot, sem):
        pltpu.make_async_copy(
            fw_hbm.at[selv.at[pl.ds(g * _GRP * _K, _GRP * _K)]],
            gbuf.at[slot], sem).wait()

    def _issue_next(g, slot, sem):
        @pl.when(g + _RING < _NG)
        def _():
            _issue_rows(g + _RING, slot, sem)

    def _group_sums(g, slot):
        sums = []
        for sg in range(_GRP):
            b = lax.rem(g * _GRP + sg, jnp.int32(_CB))
            for c in range(_E // 16):
                acc = gbuf[slot, sg * _K, pl.ds(16 * c, 16)]
                for i in range(1, _K):
                    acc = acc + gbuf[slot, sg * _K + i, pl.ds(16 * c, 16)]
                sums.append((b, c, acc))
        return sums

    def _reduce_acc(g, slot):
        for b, c, acc in _group_sums(g, slot):
            outv[b, pl.ds(16 * c, 16)] = (
                outv[b, pl.ds(16 * c, 16)] + _SCALE * acc
            )

    def _reduce_relu(g, slot):
        for b, c, acc in _group_sums(g, slot):
            outv[b, pl.ds(16 * c, 16)] = jnp.maximum(
                outv[b, pl.ds(16 * c, 16)] + _SCALE * acc, 0.0
            )

    with jax.named_scope("sc_gather"):
        for t in range(_RING):
            _issue_rows(t, t, sems[t])

        def _acc_body(gg, carry):
            for t in range(_RING):
                g = _RING * gg + t
                _wait_rows(g, t, sems[t])
                _reduce_acc(g, t)
                _issue_next(g, t, sems[t])
            return carry

        lax.fori_loop(0, (2 * _NG // 3) // _RING, _acc_body, 0)

        def _final_body(gg, carry):
            for t in range(_RING):
                g = 2 * _NG // 3 + _RING * gg + t
                _wait_rows(g, t, sems[t])
                _reduce_relu(g, t)
                _issue_next(g, t, sems[t])
            return carry

        lax.fori_loop(0, (_NG // 3) // _RING, _final_body, 0)

    with jax.named_scope("sc_tail"):
        pltpu.sync_copy(outv, out_hbm.at[pl.ds(base, _CB), :])


def kernel(nodes, labels, neigh_index, features, w_label, W):
    del labels
    w_pad = jnp.pad(W, ((0, 0), (0, _EP - _E)))
    fw, fscore = _tc_precompute(features, w_pad, w_label)
    fw64 = fw.reshape(2 * _NP, _E)
    neigh_flat = neigh_index.reshape(_R * _B * _D).astype(jnp.int32)
    return _sc_agg(nodes.astype(jnp.int32), neigh_flat, fw64, fscore)


# speedup vs baseline: 1.1278x; 1.1278x over previous
import functools

import jax
import jax.numpy as jnp
from jax import lax
from jax.experimental import pallas as pl
from jax.experimental.pallas import tpu as pltpu
from jax.experimental.pallas import tpu_sc as plsc

_N = 100000
_NP = 100352
_F = 128
_E = 64
_EP = 128
_B = 4096
_D = 32
_R = 3
_K = 16
_SCALE = 0.5 / _K

_NW = 32
_CB = _B // _NW
_ROWS = _R * _CB
_NBLK = _ROWS // 16
_GRP = 4
_NG = _ROWS // _GRP
_RING = 4
_TS = _ROWS + 1
_SG = 128
_NSG = _ROWS * _D // _SG


def _batcher_pairs(n):
    pairs = []

    def merge(lo, m, r):
        step = r * 2
        if step < m:
            merge(lo, m, step)
            merge(lo + r, m, step)
            for i in range(lo + r, lo + m - r, step):
                pairs.append((i, i + r))
        else:
            pairs.append((lo, lo + r))

    def sort(lo, m):
        if m > 1:
            h = m // 2
            sort(lo, h)
            sort(lo + h, h)
            merge(lo, m, 1)

    sort(0, n)
    return pairs


_CE_PAIRS = _batcher_pairs(_K)


def _tc_body(f_ref, w_ref, wl_ref, fw_ref, fs_ref):
    f = f_ref[...]
    fw_ref[...] = jnp.dot(f, w_ref[...], preferred_element_type=jnp.float32)
    i = pl.program_id(0)
    fs = jnp.dot(f, wl_ref[...], preferred_element_type=jnp.float32)[:, 0]
    fs_ref[pl.ds(i * _TC_BLK, _TC_BLK)] = fs


_TC_BLK = 2048
_tc_precompute = pl.pallas_call(
    _tc_body,
    grid=(_NP // _TC_BLK,),
    in_specs=[
        pl.BlockSpec((_TC_BLK, _F), lambda i: (i, 0)),
        pl.BlockSpec((_F, _EP), lambda i: (0, 0)),
        pl.BlockSpec((_F, 1), lambda i: (0, 0)),
    ],
    out_specs=[
        pl.BlockSpec((_TC_BLK, _EP), lambda i: (i, 0)),
        pl.BlockSpec((_NP,), lambda i: (0,)),
    ],
    out_shape=[
        jax.ShapeDtypeStruct((_NP, _EP), jnp.float32),
        jax.ShapeDtypeStruct((_NP,), jnp.float32),
    ],
)


_mesh = plsc.VectorSubcoreMesh(
    core_axis_name="c", subcore_axis_name="s", num_cores=2, num_subcores=16
)


@functools.partial(
    pl.kernel,
    out_type=jax.ShapeDtypeStruct((_B, _E), jnp.float32),
    mesh=_mesh,
    compiler_params=pltpu.CompilerParams(
        needs_layout_passes=False, use_tc_tiling_on_sc=False
    ),
    scratch_types=[
        pltpu.VMEM((_CB,), jnp.int32),
        pltpu.VMEM((_CB,), jnp.int32),
        pltpu.VMEM((_CB,), jnp.float32),
        pltpu.VMEM((_ROWS * _D,), jnp.int32),
        pltpu.VMEM((_ROWS * _D,), jnp.float32),
        pltpu.VMEM((_ROWS * _K,), jnp.int32),
        pltpu.VMEM((_D * _TS,), jnp.int32),
        pltpu.VMEM((_D * _TS,), jnp.float32),
        pltpu.VMEM((_CB, _E), jnp.float32),
        pltpu.VMEM((_RING, _GRP * _K, _E), jnp.float32),
        pltpu.SemaphoreType.DMA,
        pltpu.SemaphoreType.DMA,
        pltpu.SemaphoreType.DMA,
        pltpu.SemaphoreType.DMA,
        pltpu.SemaphoreType.DMA,
        pltpu.SemaphoreType.DMA,
        pltpu.SemaphoreType.DMA,
        pltpu.SemaphoreType.DMA,
        pltpu.SemaphoreType.DMA,
    ],
)
def _sc_agg(nodes_hbm, neighf_hbm, fw_hbm, fs_hbm, out_hbm,
            nodesv, nodes2v, csv, idxv, nsv, selv, idxT, nsT, outv, gbuf,
            sem0, semA, semB, semC, semD, semE, semF, semG, semH):
    wid = lax.axis_index("s") * 2 + lax.axis_index("c")
    base = wid * _CB

    pltpu.sync_copy(nodes_hbm.at[pl.ds(base, _CB)], nodesv)
    for r in range(_R):
        pltpu.sync_copy(
            neighf_hbm.at[pl.ds(r * _B * _D + base * _D, _CB * _D)],
            idxv.at[pl.ds(r * _CB * _D, _CB * _D)],
        )

    cs_cp = pltpu.async_copy(fs_hbm.at[nodesv], csv, sem0)
    def _dbl_body(t, carry):
        v = nodesv[pl.ds(t * 16, 16)]
        nodes2v[pl.ds(t * 16, 16)] = v + v
        return carry
    lax.fori_loop(0, _CB // 16, _dbl_body, 0)
    ch_cp0 = pltpu.async_copy(
        fw_hbm.at[nodes2v.at[pl.ds(0, _CB // 2)]], gbuf.at[0], sem0)
    ch_cp1 = pltpu.async_copy(
        fw_hbm.at[nodes2v.at[pl.ds(_CB // 2, _CB // 2)]], gbuf.at[1], sem0)

    with jax.named_scope("sc_scores"):
        qs = _ROWS * _D // 4
        ns_cps = [
            pltpu.async_copy(
                fs_hbm.at[idxv.at[pl.ds(q * qs, qs)]],
                nsv.at[pl.ds(q * qs, qs)], sem)
            for q, sem in enumerate((semA, semB, semC, semD))
        ]
        cs_cp.wait()
        for cp in ns_cps:
            cp.wait()

        def _tr_body(rr, carry):
            base32 = rr * _D
            for h in range(2):
                iv = idxv[pl.ds(base32 + h * 16, 16)]
                sv = nsv[pl.ds(base32 + h * 16, 16)]
                addr = (h * 16 + lax.iota(jnp.int32, 16)) * _TS + rr
                plsc.store_scatter(idxT, [addr], iv)
                plsc.store_scatter(nsT, [addr], sv)
            return carry

        lax.fori_loop(0, _ROWS, _tr_body, 0)

    def _sel_body(blk, carry):
        rr = blk * 16 + lax.iota(jnp.int32, 16)
        b = lax.rem(rr, jnp.int32(_CB))
        cs = plsc.load_gather(csv, [b])
        rowbase = blk * 16
        base16 = rr * _K
        keys = []
        vals = []
        for j in range(_D):
            s = nsT[pl.ds(j * _TS + rowbase, 16)]
            keys.append(jnp.abs(s - cs))
            vals.append(idxT[pl.ds(j * _TS + rowbase, 16)])
        for half in (0, _K):
            for (i, j) in _CE_PAIRS:
                a, bb = half + i, half + j
                c = keys[a] <= keys[bb]
                ka, kb = keys[a], keys[bb]
                va, vb = vals[a], vals[bb]
                keys[a] = jnp.where(c, ka, kb)
                keys[bb] = jnp.where(c, kb, ka)
                vals[a] = jnp.where(c, va, vb)
                vals[bb] = jnp.where(c, vb, va)
        for j in range(_K):
            a, bb = j, _K + (_K - 1 - j)
            c = keys[a] <= keys[bb]
            sel = jnp.where(c, vals[a], vals[bb])
            plsc.store_scatter(selv, [base16 + j], sel + sel)
        return carry

    with jax.named_scope("sc_select"):
        lax.fori_loop(0, _NBLK, _sel_body, 0)
        ch_cp0.wait()
        ch_cp1.wait()
        def _ch_body(b, carry):
            for c in range(_E // 16):
                outv[b, pl.ds(16 * c, 16)] = gbuf[0, b, pl.ds(16 * c, 16)]
                outv[b + _CB // 2, pl.ds(16 * c, 16)] = (
                    gbuf[1, b, pl.ds(16 * c, 16)])
            return carry
        lax.fori_loop(0, _CB // 2, _ch_body, 0)

    sems = (semA, semB, semC, semD, semE, semF, semG, semH)

    def _issue_rows(g, slot, sem):
        pltpu.async_copy(
            fw_hbm.at[selv.at[pl.ds(g * _GRP * _K, _GRP * _K)]],
            gbuf.at[slot], sem)

    def _wait_rows(g, slot, sem):
        pltpu.make_async_copy(
            fw_hbm.at[selv.at[pl.ds(g * _GRP * _K, _GRP * _K)]],
            gbuf.at[slot], sem).wait()

    def _issue_next(g, slot, sem):
        @pl.when(g + _RING < _NG)
        def _():
            _issue_rows(g + _RING, slot, sem)

    def _group_sums(g, slot):
        sums = []
        for sg in range(_GRP):
            b = lax.rem(g * _GRP + sg, jnp.int32(_CB))
            for c in range(_E // 16):
                acc = gbuf[slot, sg * _K, pl.ds(16 * c, 16)]
                for i in range(1, _K):
                    acc = acc + gbuf[slot, sg * _K + i, pl.ds(16 * c, 16)]
                sums.append((b, c, acc))
        return sums

    def _reduce_acc(g, slot):
        for b, c, acc in _group_sums(g, slot):
            outv[b, pl.ds(16 * c, 16)] = (
                outv[b, pl.ds(16 * c, 16)] + _SCALE * acc
            )

    def _reduce_relu(g, slot):
        for b, c, acc in _group_sums(g, slot):
            outv[b, pl.ds(16 * c, 16)] = jnp.maximum(
                outv[b, pl.ds(16 * c, 16)] + _SCALE * acc, 0.0
            )

    with jax.named_scope("sc_gather"):
        for t in range(_RING):
            _issue_rows(t, t, sems[t])

        def _acc_body(gg, carry):
            for t in range(_RING):
                g = _RING * gg + t
                _wait_rows(g, t, sems[t])
                _reduce_acc(g, t)
                _issue_next(g, t, sems[t])
            return carry

        lax.fori_loop(0, (2 * _NG // 3) // _RING, _acc_body, 0)

        def _final_body(gg, carry):
            for t in range(_RING):
                g = 2 * _NG // 3 + _RING * gg + t
                _wait_rows(g, t, sems[t])
                _reduce_relu(g, t)
                _issue_next(g, t, sems[t])
            return carry

        lax.fori_loop(0, (_NG // 3) // _RING, _final_body, 0)

    with jax.named_scope("sc_tail"):
        pltpu.sync_copy(outv, out_hbm.at[pl.ds(base, _CB), :])


def kernel(nodes, labels, neigh_index, features, w_label, W):
    del labels
    w_pad = jnp.pad(W, ((0, 0), (0, _EP - _E)))
    fw, fscore = _tc_precompute(features, w_pad, w_label)
    fw64 = fw.reshape(2 * _NP, _E)
    neigh_flat = neigh_index.reshape(_R * _B * _D).astype(jnp.int32)
    return _sc_agg(nodes.astype(jnp.int32), neigh_flat, fw64, fscore)
